# baseline (speedup 1.0000x reference)
import functools

import jax
import jax.numpy as jnp
from jax import lax
from jax.experimental import pallas as pl
from jax.experimental.pallas import tpu as pltpu
from jax.experimental.pallas import tpu_sc as plsc

NC, NS, L = 2, 16, 16
NW = NC * NS


def _node_tables(node_emb, w_s, w_r, b_msg, *, bn):
    n, d = node_emb.shape
    msg = w_s.shape[1]

    def body(x_ref, ws_ref, wr_ref, b_ref, a_ref, bb_ref):
        x = x_ref[...]
        a_ref[...] = jnp.dot(x, ws_ref[...], preferred_element_type=jnp.float32)
        bb_ref[...] = (
            jnp.dot(x, wr_ref[...], preferred_element_type=jnp.float32) + b_ref[...]
        )

    return pl.pallas_call(
        body,
        grid=(n // bn,),
        in_specs=[
            pl.BlockSpec((bn, d), lambda i: (i, 0)),
            pl.BlockSpec((d, msg), lambda i: (0, 0)),
            pl.BlockSpec((d, msg), lambda i: (0, 0)),
            pl.BlockSpec((1, msg), lambda i: (0, 0)),
        ],
        out_specs=[
            pl.BlockSpec((bn, msg), lambda i: (i, 0)),
            pl.BlockSpec((bn, msg), lambda i: (i, 0)),
        ],
        out_shape=[
            jax.ShapeDtypeStruct((n, msg), jnp.float32),
            jax.ShapeDtypeStruct((n, msg), jnp.float32),
        ],
    )(node_emb, w_s, w_r, b_msg)


def _edge_proj(edge_emb, w_e, *, be, e_out):
    e, de = edge_emb.shape
    msg = w_e.shape[1]
    nreal = e // be

    def body(x_ref, w_ref, o_ref):
        o_ref[...] = jnp.dot(x_ref[...], w_ref[...], preferred_element_type=jnp.float32)

    return pl.pallas_call(
        body,
        grid=(e_out // be,),
        in_specs=[
            pl.BlockSpec((be, de), lambda i: (jnp.minimum(i, nreal - 1), 0)),
            pl.BlockSpec((de, msg), lambda i: (0, 0)),
        ],
        out_specs=pl.BlockSpec((be, msg), lambda i: (i, 0)),
        out_shape=jax.ShapeDtypeStruct((e_out, msg), jnp.float32),
    )(edge_emb, w_e)


def _sc_messages(a_tab, b_flat, ce_flat, snd_p, rcv_p, bnd, *, n, msg, k, wr):
    nvec = msg // L
    last_rows = n - wr * (NW - 1)
    assert 0 < last_rows <= wr
    mesh = plsc.VectorSubcoreMesh(core_axis_name="c", subcore_axis_name="s")

    @functools.partial(
        pl.kernel,
        out_type=jax.ShapeDtypeStruct((n * msg,), jnp.float32),
        mesh=mesh,
        scratch_types=[
            pltpu.VMEM(((wr + 1) * msg,), jnp.float32),
            pltpu.VMEM((wr * msg,), jnp.float32),
            pltpu.VMEM((k, msg), jnp.float32),
            pltpu.VMEM((k * msg,), jnp.float32),
            pltpu.VMEM((k,), jnp.int32),
            pltpu.VMEM((k + L,), jnp.int32),
            pltpu.VMEM((k + L,), jnp.int32),
            pltpu.VMEM((k, msg), jnp.float32),
            pltpu.VMEM((k * msg,), jnp.float32),
            pltpu.VMEM((k,), jnp.int32),
            pltpu.VMEM((k + L,), jnp.int32),
            pltpu.VMEM((k + L,), jnp.int32),
            pltpu.VMEM((48,), jnp.int32),
            pltpu.SemaphoreType.DMA,
            pltpu.SemaphoreType.DMA,
            pltpu.SemaphoreType.DMA,
            pltpu.SemaphoreType.DMA,
            pltpu.SemaphoreType.DMA,
            pltpu.SemaphoreType.DMA,
        ],
    )
    def sc_kernel(
        a_hbm, b_hbm, ce_hbm, snd_hbm, rcv_hbm, bnd_hbm, out_hbm,
        outbuf, bstripe,
        ba0, bc0, is0, ir0, irc0, ba1, bc1, is1, ir1, irc1, bndv,
        sa0, sc0, si0, sa1, sc1, si1,
    ):
        cid = lax.axis_index("c")
        sid = lax.axis_index("s")
        wid = sid * NC + cid
        ba, bc = (ba0, ba1), (bc0, bc1)
        isx, irx, irc = (is0, is1), (ir0, ir1), (irc0, irc1)
        sa, sc, si = (sa0, sa1), (sc0, sc1), (si0, si1)

        pltpu.sync_copy(bnd_hbm, bndv)
        e0 = bndv[pl.ds(wid, L)][0]
        e1 = bndv[pl.ds(wid + 1, L)][0]
        base_row = wid * wr
        ec0 = (e0 // 8) * 8
        nch2 = (e1 - ec0 + 2 * k - 1) // (2 * k)

        def idx_start(c, b):
            off = ec0 + c * k
            pltpu.async_copy(snd_hbm.at[pl.ds(off, k)], isx[b], si[b])
            pltpu.async_copy(rcv_hbm.at[pl.ds(off, k)], irx[b].at[pl.ds(0, k)], si[b])

        def idx_wait(c, b):
            off = ec0 + c * k
            pltpu.make_async_copy(snd_hbm.at[pl.ds(off, k)], isx[b], si[b]).wait()
            pltpu.make_async_copy(
                rcv_hbm.at[pl.ds(off, k)], irx[b].at[pl.ds(0, k)], si[b]
            ).wait()

        def gather_start(c, b):
            pltpu.async_copy(a_hbm.at[isx[b]], ba[b], sa[b])
            pltpu.async_copy(
                ce_hbm.at[pl.ds((ec0 + c * k) * msg, k * msg)], bc[b], sc[b]
            )

        def gather_wait(c, b):
            pltpu.make_async_copy(a_hbm.at[isx[b]], ba[b], sa[b]).wait()
            pltpu.make_async_copy(
                ce_hbm.at[pl.ds((ec0 + c * k) * msg, k * msg)], bc[b], sc[b]
            ).wait()

        def _prime():
            idx_start(0, 0)
            idx_start(1, 1)

        pl.when(nch2 > 0)(_prime)

        def _bs_full():
            pltpu.async_copy(
                b_hbm.at[pl.ds(base_row * msg, wr * msg)],
                bstripe.at[pl.ds(0, wr * msg)],
                sa0,
            )

        def _bs_last():
            pltpu.async_copy(
                b_hbm.at[pl.ds(base_row * msg, last_rows * msg)],
                bstripe.at[pl.ds(0, last_rows * msg)],
                sa0,
            )

        pl.when(wid < NW - 1)(_bs_full)
        pl.when(wid == NW - 1)(_bs_last)

        def zvec(i, carry):
            outbuf[pl.ds(i * L, L)] = jnp.zeros((L,), jnp.float32)
            return carry

        lax.fori_loop(0, (wr + 1) * nvec, zvec, 0)

        def _bsw_full():
            pltpu.make_async_copy(
                b_hbm.at[pl.ds(base_row * msg, wr * msg)],
                bstripe.at[pl.ds(0, wr * msg)],
                sa0,
            ).wait()

        def _bsw_last():
            pltpu.make_async_copy(
                b_hbm.at[pl.ds(base_row * msg, last_rows * msg)],
                bstripe.at[pl.ds(0, last_rows * msg)],
                sa0,
            ).wait()

        pl.when(wid < NW - 1)(_bsw_full)
        pl.when(wid == NW - 1)(_bsw_last)

        def _first_gathers():
            idx_wait(0, 0)
            gather_start(0, 0)

        pl.when(nch2 > 0)(_first_gathers)

        zero16 = jnp.zeros((L,), jnp.float32)
        init = (jnp.array(-1, jnp.int32), (zero16,) * nvec)

        def process(c, b, carry, more):
            gather_wait(c, b)
            for j in range((k + L) // L):
                irc[b][pl.ds(j * L, L)] = irx[b][pl.ds(j * L, L)]
            pl.when(more)(lambda: idx_start(c + 2, b))

            def _next_gathers():
                idx_wait(c + 1, 1 - b)
                gather_start(c + 1, 1 - b)

            if b == 0:
                _next_gathers()
            else:
                pl.when(more)(_next_gathers)

            def one_edge(i, ecar):
                r_cur, accs = ecar
                e_abs = ec0 + c * k + i
                in_r = jnp.logical_and(e_abs >= e0, e_abs < e1)
                r_i = irc[b][pl.ds(i, L)][0]
                same = r_i == r_cur

                def flush():
                    rl_old = jnp.clip(
                        jnp.where(r_cur >= 0, r_cur - base_row, wr), 0, wr
                    )
                    for j in range(nvec):
                        outbuf[pl.ds(rl_old * msg + j * L, L)] = accs[j]

                pl.when(jnp.logical_and(in_r, jnp.logical_not(same)))(flush)
                r_new = jnp.where(in_r, r_i, r_cur)
                rl = jnp.clip(r_new - base_row, 0, wr - 1)
                new_accs = []
                for j in range(nvec):
                    m = jnp.maximum(
                        bc[b][pl.ds(i * msg + j * L, L)]
                        + ba[b][i, pl.ds(j * L, L)]
                        + bstripe[pl.ds(rl * msg + j * L, L)],
                        0.0,
                    )
                    a_prev = jnp.where(same, accs[j], 0.0)
                    new_accs.append(jnp.where(in_r, a_prev + m, accs[j]))
                return (r_new, tuple(new_accs))

            return lax.fori_loop(0, k, one_edge, carry)

        def pair(g, carry):
            more = g < nch2 - 1
            carry = process(2 * g, 0, carry, more)
            carry = process(2 * g + 1, 1, carry, more)
            return carry

        r_fin, accs_fin = lax.fori_loop(0, nch2, pair, init)
        rl_fin = jnp.clip(jnp.where(r_fin >= 0, r_fin - base_row, wr), 0, wr)
        for j in range(nvec):
            outbuf[pl.ds(rl_fin * msg + j * L, L)] = accs_fin[j]

        def _store_full():
            pltpu.sync_copy(
                outbuf.at[pl.ds(0, wr * msg)],
                out_hbm.at[pl.ds(base_row * msg, wr * msg)],
            )

        def _store_last():
            pltpu.sync_copy(
                outbuf.at[pl.ds(0, last_rows * msg)],
                out_hbm.at[pl.ds(base_row * msg, last_rows * msg)],
            )

        pl.when(wid < NW - 1)(_store_full)
        pl.when(wid == NW - 1)(_store_last)

    return sc_kernel(a_tab, b_flat, ce_flat, snd_p, rcv_p, bnd)


def _update(node_emb, msgs, w_upd, b_upd, *, bn):
    n, d = node_emb.shape
    msg = msgs.shape[-1]

    def body(x_ref, p_ref, w_ref, b_ref, o_ref):
        x = x_ref[...]
        p = p_ref[...]
        w1 = w_ref[:d]
        w2 = w_ref[d:]
        o_ref[...] = jnp.tanh(
            jnp.dot(x, w1, preferred_element_type=jnp.float32)
            + jnp.dot(p, w2, preferred_element_type=jnp.float32)
            + b_ref[...]
        )

    return pl.pallas_call(
        body,
        grid=(n // bn,),
        in_specs=[
            pl.BlockSpec((bn, d), lambda i: (i, 0)),
            pl.BlockSpec((bn, msg), lambda i: (i, 0)),
            pl.BlockSpec((d + msg, d), lambda i: (0, 0)),
            pl.BlockSpec((1, d), lambda i: (0, 0)),
        ],
        out_specs=pl.BlockSpec((bn, d), lambda i: (i, 0)),
        out_shape=jax.ShapeDtypeStruct((n, d), jnp.float32),
    )(node_emb, msgs, w_upd, b_upd)


def kernel(node_emb, edge_emb, senders, receivers, W_msg, b_msg, W_upd, b_upd):
    n, d = node_emb.shape
    e, de = edge_emb.shape
    msg = W_msg.shape[1]
    w_e = W_msg[:de]
    w_s = W_msg[de : de + d]
    w_r = W_msg[de + d :]

    k = 96
    wr = -(-n // NW)
    pad = 2000

    snd_p = jnp.concatenate([senders, jnp.zeros((pad,), jnp.int32)])
    rcv_p = jnp.concatenate([receivers, jnp.zeros((pad,), jnp.int32)])

    a_tab, b_tab = _node_tables(node_emb, w_s, w_r, b_msg.reshape(1, msg), bn=2000)
    ce = _edge_proj(edge_emb, w_e, be=2000, e_out=e + pad)

    bnd = jnp.searchsorted(
        receivers, (wr * jnp.arange(NW + 1)).astype(jnp.int32)
    ).astype(jnp.int32)
    bnd = jnp.concatenate([bnd, jnp.full((48 - (NW + 1),), e, jnp.int32)])

    msgs_flat = _sc_messages(
        a_tab, b_tab.reshape(-1), ce.reshape(-1), snd_p, rcv_p, bnd,
        n=n, msg=msg, k=k, wr=wr,
    )
    return _update(node_emb, msgs_flat.reshape(n, msg), W_upd, b_upd.reshape(1, d), bn=1000)


# speedup vs baseline: 1.0045x; 1.0045x over previous
import functools

import jax
import jax.numpy as jnp
from jax import lax
from jax.experimental import pallas as pl
from jax.experimental.pallas import tpu as pltpu
from jax.experimental.pallas import tpu_sc as plsc

NC, NS, L = 2, 16, 16
NW = NC * NS


def _node_tables(node_emb, w_s, w_r, b_msg, *, bn):
    n, d = node_emb.shape
    msg = w_s.shape[1]

    def body(x_ref, ws_ref, wr_ref, b_ref, a_ref, bb_ref):
        x = x_ref[...]
        a_ref[...] = jnp.dot(x, ws_ref[...], preferred_element_type=jnp.float32)
        bb_ref[...] = (
            jnp.dot(x, wr_ref[...], preferred_element_type=jnp.float32) + b_ref[...]
        )

    return pl.pallas_call(
        body,
        grid=(n // bn,),
        in_specs=[
            pl.BlockSpec((bn, d), lambda i: (i, 0)),
            pl.BlockSpec((d, msg), lambda i: (0, 0)),
            pl.BlockSpec((d, msg), lambda i: (0, 0)),
            pl.BlockSpec((1, msg), lambda i: (0, 0)),
        ],
        out_specs=[
            pl.BlockSpec((bn, msg), lambda i: (i, 0)),
            pl.BlockSpec((bn, msg), lambda i: (i, 0)),
        ],
        out_shape=[
            jax.ShapeDtypeStruct((n, msg), jnp.float32),
            jax.ShapeDtypeStruct((n, msg), jnp.float32),
        ],
    )(node_emb, w_s, w_r, b_msg)


def _edge_proj(edge_emb, w_e, *, be, e_out):
    e, de = edge_emb.shape
    msg = w_e.shape[1]
    nreal = e // be

    def body(x_ref, w_ref, o_ref):
        o_ref[...] = jnp.dot(x_ref[...], w_ref[...], preferred_element_type=jnp.float32)

    return pl.pallas_call(
        body,
        grid=(e_out // be,),
        in_specs=[
            pl.BlockSpec((be, de), lambda i: (jnp.minimum(i, nreal - 1), 0)),
            pl.BlockSpec((de, msg), lambda i: (0, 0)),
        ],
        out_specs=pl.BlockSpec((be, msg), lambda i: (i, 0)),
        out_shape=jax.ShapeDtypeStruct((e_out, msg), jnp.float32),
    )(edge_emb, w_e)


def _sc_messages(a_tab, b_flat, ce_flat, snd_p, rcv_p, bnd, *, n, msg, k, wr):
    nvec = msg // L
    last_rows = n - wr * (NW - 1)
    assert 0 < last_rows <= wr
    mesh = plsc.VectorSubcoreMesh(core_axis_name="c", subcore_axis_name="s")

    @functools.partial(
        pl.kernel,
        out_type=jax.ShapeDtypeStruct((n * msg,), jnp.float32),
        mesh=mesh,
        scratch_types=[
            pltpu.VMEM(((wr + 1) * msg,), jnp.float32),
            pltpu.VMEM((wr * msg,), jnp.float32),
            pltpu.VMEM((k, msg), jnp.float32),
            pltpu.VMEM((k * msg,), jnp.float32),
            pltpu.VMEM((k,), jnp.int32),
            pltpu.VMEM((k + L,), jnp.int32),
            pltpu.VMEM((k + L,), jnp.int32),
            pltpu.VMEM((k, msg), jnp.float32),
            pltpu.VMEM((k * msg,), jnp.float32),
            pltpu.VMEM((k,), jnp.int32),
            pltpu.VMEM((k + L,), jnp.int32),
            pltpu.VMEM((k + L,), jnp.int32),
            pltpu.VMEM((48,), jnp.int32),
            pltpu.SemaphoreType.DMA,
            pltpu.SemaphoreType.DMA,
            pltpu.SemaphoreType.DMA,
            pltpu.SemaphoreType.DMA,
            pltpu.SemaphoreType.DMA,
            pltpu.SemaphoreType.DMA,
        ],
    )
    def sc_kernel(
        a_hbm, b_hbm, ce_hbm, snd_hbm, rcv_hbm, bnd_hbm, out_hbm,
        outbuf, bstripe,
        ba0, bc0, is0, ir0, irc0, ba1, bc1, is1, ir1, irc1, bndv,
        sa0, sc0, si0, sa1, sc1, si1,
    ):
        cid = lax.axis_index("c")
        sid = lax.axis_index("s")
        wid = sid * NC + cid
        ba, bc = (ba0, ba1), (bc0, bc1)
        isx, irx, irc = (is0, is1), (ir0, ir1), (irc0, irc1)
        sa, sc, si = (sa0, sa1), (sc0, sc1), (si0, si1)

        pltpu.sync_copy(bnd_hbm, bndv)
        e0 = bndv[pl.ds(wid, L)][0]
        e1 = bndv[pl.ds(wid + 1, L)][0]
        base_row = wid * wr
        ec0 = (e0 // 8) * 8
        nch2 = (e1 - ec0 + 2 * k - 1) // (2 * k)

        def idx_start(c, b):
            off = ec0 + c * k
            pltpu.async_copy(snd_hbm.at[pl.ds(off, k)], isx[b], si[b])
            pltpu.async_copy(rcv_hbm.at[pl.ds(off, k)], irx[b].at[pl.ds(0, k)], si[b])

        def idx_wait(c, b):
            off = ec0 + c * k
            pltpu.make_async_copy(snd_hbm.at[pl.ds(off, k)], isx[b], si[b]).wait()
            pltpu.make_async_copy(
                rcv_hbm.at[pl.ds(off, k)], irx[b].at[pl.ds(0, k)], si[b]
            ).wait()

        def gather_start(c, b):
            pltpu.async_copy(a_hbm.at[isx[b]], ba[b], sa[b])
            pltpu.async_copy(
                ce_hbm.at[pl.ds((ec0 + c * k) * msg, k * msg)], bc[b], sc[b]
            )

        def gather_wait(c, b):
            pltpu.make_async_copy(a_hbm.at[isx[b]], ba[b], sa[b]).wait()
            pltpu.make_async_copy(
                ce_hbm.at[pl.ds((ec0 + c * k) * msg, k * msg)], bc[b], sc[b]
            ).wait()

        def _prime():
            idx_start(0, 0)
            idx_start(1, 1)

        pl.when(nch2 > 0)(_prime)

        def _bs_full():
            pltpu.async_copy(
                b_hbm.at[pl.ds(base_row * msg, wr * msg)],
                bstripe.at[pl.ds(0, wr * msg)],
                sa0,
            )

        def _bs_last():
            pltpu.async_copy(
                b_hbm.at[pl.ds(base_row * msg, last_rows * msg)],
                bstripe.at[pl.ds(0, last_rows * msg)],
                sa0,
            )

        pl.when(wid < NW - 1)(_bs_full)
        pl.when(wid == NW - 1)(_bs_last)

        def zvec(i, carry):
            outbuf[pl.ds(i * L, L)] = jnp.zeros((L,), jnp.float32)
            return carry

        lax.fori_loop(0, (wr + 1) * nvec, zvec, 0)

        def _bsw_full():
            pltpu.make_async_copy(
                b_hbm.at[pl.ds(base_row * msg, wr * msg)],
                bstripe.at[pl.ds(0, wr * msg)],
                sa0,
            ).wait()

        def _bsw_last():
            pltpu.make_async_copy(
                b_hbm.at[pl.ds(base_row * msg, last_rows * msg)],
                bstripe.at[pl.ds(0, last_rows * msg)],
                sa0,
            ).wait()

        pl.when(wid < NW - 1)(_bsw_full)
        pl.when(wid == NW - 1)(_bsw_last)

        def _first_gathers():
            idx_wait(0, 0)
            gather_start(0, 0)

        pl.when(nch2 > 0)(_first_gathers)

        zero16 = jnp.zeros((L,), jnp.float32)
        init = (jnp.array(-1, jnp.int32), (zero16,) * nvec)

        def process(c, b, carry, more):
            gather_wait(c, b)
            for j in range((k + L) // L):
                irc[b][pl.ds(j * L, L)] = irx[b][pl.ds(j * L, L)]
            pl.when(more)(lambda: idx_start(c + 2, b))

            def _next_gathers():
                idx_wait(c + 1, 1 - b)
                gather_start(c + 1, 1 - b)

            if b == 0:
                _next_gathers()
            else:
                pl.when(more)(_next_gathers)

            def one_edge(i, ecar):
                r_cur, accs = ecar
                e_abs = ec0 + c * k + i
                in_r = jnp.logical_and(e_abs >= e0, e_abs < e1)
                r_i = irc[b][pl.ds(i, L)][0]
                same = r_i == r_cur

                def flush():
                    rl_old = jnp.clip(
                        jnp.where(r_cur >= 0, r_cur - base_row, wr), 0, wr
                    )
                    for j in range(nvec):
                        outbuf[pl.ds(rl_old * msg + j * L, L)] = accs[j]

                pl.when(jnp.logical_and(in_r, jnp.logical_not(same)))(flush)
                r_new = jnp.where(in_r, r_i, r_cur)
                rl = jnp.clip(r_new - base_row, 0, wr - 1)
                new_accs = []
                for j in range(nvec):
                    m = jnp.maximum(
                        bc[b][pl.ds(i * msg + j * L, L)]
                        + ba[b][i, pl.ds(j * L, L)]
                        + bstripe[pl.ds(rl * msg + j * L, L)],
                        0.0,
                    )
                    a_prev = jnp.where(same, accs[j], 0.0)
                    new_accs.append(jnp.where(in_r, a_prev + m, accs[j]))
                return (r_new, tuple(new_accs))

            return lax.fori_loop(0, k, one_edge, carry)

        def pair(g, carry):
            more = g < nch2 - 1
            carry = process(2 * g, 0, carry, more)
            carry = process(2 * g + 1, 1, carry, more)
            return carry

        r_fin, accs_fin = lax.fori_loop(0, nch2, pair, init)
        rl_fin = jnp.clip(jnp.where(r_fin >= 0, r_fin - base_row, wr), 0, wr)
        for j in range(nvec):
            outbuf[pl.ds(rl_fin * msg + j * L, L)] = accs_fin[j]

        def _store_full():
            pltpu.sync_copy(
                outbuf.at[pl.ds(0, wr * msg)],
                out_hbm.at[pl.ds(base_row * msg, wr * msg)],
            )

        def _store_last():
            pltpu.sync_copy(
                outbuf.at[pl.ds(0, last_rows * msg)],
                out_hbm.at[pl.ds(base_row * msg, last_rows * msg)],
            )

        pl.when(wid < NW - 1)(_store_full)
        pl.when(wid == NW - 1)(_store_last)

    return sc_kernel(a_tab, b_flat, ce_flat, snd_p, rcv_p, bnd)


def _update(node_emb, msgs, w_upd, b_upd, *, bn):
    n, d = node_emb.shape
    msg = msgs.shape[-1]

    def body(x_ref, p_ref, w_ref, b_ref, o_ref):
        x = x_ref[...]
        p = p_ref[...]
        w1 = w_ref[:d]
        w2 = w_ref[d:]
        o_ref[...] = jnp.tanh(
            jnp.dot(x, w1, preferred_element_type=jnp.float32)
            + jnp.dot(p, w2, preferred_element_type=jnp.float32)
            + b_ref[...]
        )

    return pl.pallas_call(
        body,
        grid=(n // bn,),
        in_specs=[
            pl.BlockSpec((bn, d), lambda i: (i, 0)),
            pl.BlockSpec((bn, msg), lambda i: (i, 0)),
            pl.BlockSpec((d + msg, d), lambda i: (0, 0)),
            pl.BlockSpec((1, d), lambda i: (0, 0)),
        ],
        out_specs=pl.BlockSpec((bn, d), lambda i: (i, 0)),
        out_shape=jax.ShapeDtypeStruct((n, d), jnp.float32),
    )(node_emb, msgs, w_upd, b_upd)


def kernel(node_emb, edge_emb, senders, receivers, W_msg, b_msg, W_upd, b_upd):
    n, d = node_emb.shape
    e, de = edge_emb.shape
    msg = W_msg.shape[1]
    w_e = W_msg[:de]
    w_s = W_msg[de : de + d]
    w_r = W_msg[de + d :]

    k = 64
    wr = -(-n // NW)
    pad = 2000

    snd_p = jnp.concatenate([senders, jnp.zeros((pad,), jnp.int32)])
    rcv_p = jnp.concatenate([receivers, jnp.zeros((pad,), jnp.int32)])

    a_tab, b_tab = _node_tables(node_emb, w_s, w_r, b_msg.reshape(1, msg), bn=2000)
    ce = _edge_proj(edge_emb, w_e, be=2000, e_out=e + pad)

    bnd = jnp.searchsorted(
        receivers, (wr * jnp.arange(NW + 1)).astype(jnp.int32)
    ).astype(jnp.int32)
    bnd = jnp.concatenate([bnd, jnp.full((48 - (NW + 1),), e, jnp.int32)])

    msgs_flat = _sc_messages(
        a_tab, b_tab.reshape(-1), ce.reshape(-1), snd_p, rcv_p, bnd,
        n=n, msg=msg, k=k, wr=wr,
    )
    return _update(node_emb, msgs_flat.reshape(n, msg), W_upd, b_upd.reshape(1, d), bn=1000)


# speedup vs baseline: 1.0131x; 1.0085x over previous
import functools

import jax
import jax.numpy as jnp
from jax import lax
from jax.experimental import pallas as pl
from jax.experimental.pallas import tpu as pltpu
from jax.experimental.pallas import tpu_sc as plsc

NC, NS, L = 2, 16, 16
NW = NC * NS


def _tables_and_edge_proj(node_emb, edge_emb, w_s, w_r, b_msg, w_e, *, bn, be, e_out):
    n, d = node_emb.shape
    e, de = edge_emb.shape
    msg = w_e.shape[1]
    nreal = e // be
    nblk = n // bn

    def body(x_ref, we_ref, nd_ref, ws_ref, wr_ref, b_ref, o_ref, a_ref, bb_ref):
        i = pl.program_id(0)
        o_ref[...] = jnp.dot(x_ref[...], we_ref[...], preferred_element_type=jnp.float32)

        @pl.when(i < nblk)
        def _():
            x = nd_ref[...]
            a_ref[...] = jnp.dot(x, ws_ref[...], preferred_element_type=jnp.float32)
            bb_ref[...] = (
                jnp.dot(x, wr_ref[...], preferred_element_type=jnp.float32)
                + b_ref[...]
            )

    clamp_n = lambda i: (jnp.minimum(i, nblk - 1), 0)
    return pl.pallas_call(
        body,
        grid=(e_out // be,),
        in_specs=[
            pl.BlockSpec((be, de), lambda i: (jnp.minimum(i, nreal - 1), 0)),
            pl.BlockSpec((de, msg), lambda i: (0, 0)),
            pl.BlockSpec((bn, d), clamp_n),
            pl.BlockSpec((d, msg), lambda i: (0, 0)),
            pl.BlockSpec((d, msg), lambda i: (0, 0)),
            pl.BlockSpec((1, msg), lambda i: (0, 0)),
        ],
        out_specs=[
            pl.BlockSpec((be, msg), lambda i: (i, 0)),
            pl.BlockSpec((bn, msg), clamp_n),
            pl.BlockSpec((bn, msg), clamp_n),
        ],
        out_shape=[
            jax.ShapeDtypeStruct((e_out, msg), jnp.float32),
            jax.ShapeDtypeStruct((n, msg), jnp.float32),
            jax.ShapeDtypeStruct((n, msg), jnp.float32),
        ],
    )(edge_emb, w_e, node_emb, w_s, w_r, b_msg)


def _sc_messages(a_tab, b_flat, ce_flat, snd_p, rcv_p, bnd, *, n, msg, k, wr):
    nvec = msg // L
    last_rows = n - wr * (NW - 1)
    assert 0 < last_rows <= wr
    mesh = plsc.VectorSubcoreMesh(core_axis_name="c", subcore_axis_name="s")

    @functools.partial(
        pl.kernel,
        out_type=jax.ShapeDtypeStruct((n * msg,), jnp.float32),
        mesh=mesh,
        scratch_types=[
            pltpu.VMEM(((wr + 1) * msg,), jnp.float32),
            pltpu.VMEM((wr * msg,), jnp.float32),
            pltpu.VMEM((k, msg), jnp.float32),
            pltpu.VMEM((k * msg,), jnp.float32),
            pltpu.VMEM((k,), jnp.int32),
            pltpu.VMEM((k + L,), jnp.int32),
            pltpu.VMEM((k + L,), jnp.int32),
            pltpu.VMEM((k, msg), jnp.float32),
            pltpu.VMEM((k * msg,), jnp.float32),
            pltpu.VMEM((k,), jnp.int32),
            pltpu.VMEM((k + L,), jnp.int32),
            pltpu.VMEM((k + L,), jnp.int32),
            pltpu.VMEM((48,), jnp.int32),
            pltpu.SemaphoreType.DMA,
            pltpu.SemaphoreType.DMA,
            pltpu.SemaphoreType.DMA,
            pltpu.SemaphoreType.DMA,
            pltpu.SemaphoreType.DMA,
            pltpu.SemaphoreType.DMA,
        ],
    )
    def sc_kernel(
        a_hbm, b_hbm, ce_hbm, snd_hbm, rcv_hbm, bnd_hbm, out_hbm,
        outbuf, bstripe,
        ba0, bc0, is0, ir0, irc0, ba1, bc1, is1, ir1, irc1, bndv,
        sa0, sc0, si0, sa1, sc1, si1,
    ):
        cid = lax.axis_index("c")
        sid = lax.axis_index("s")
        wid = sid * NC + cid
        ba, bc = (ba0, ba1), (bc0, bc1)
        isx, irx, irc = (is0, is1), (ir0, ir1), (irc0, irc1)
        sa, sc, si = (sa0, sa1), (sc0, sc1), (si0, si1)

        pltpu.sync_copy(bnd_hbm, bndv)
        e0 = bndv[pl.ds(wid, L)][0]
        e1 = bndv[pl.ds(wid + 1, L)][0]
        base_row = wid * wr
        ec0 = (e0 // 8) * 8
        nch2 = (e1 - ec0 + 2 * k - 1) // (2 * k)

        def idx_start(c, b):
            off = ec0 + c * k
            pltpu.async_copy(snd_hbm.at[pl.ds(off, k)], isx[b], si[b])
            pltpu.async_copy(rcv_hbm.at[pl.ds(off, k)], irx[b].at[pl.ds(0, k)], si[b])

        def idx_wait(c, b):
            off = ec0 + c * k
            pltpu.make_async_copy(snd_hbm.at[pl.ds(off, k)], isx[b], si[b]).wait()
            pltpu.make_async_copy(
                rcv_hbm.at[pl.ds(off, k)], irx[b].at[pl.ds(0, k)], si[b]
            ).wait()

        def gather_start(c, b):
            pltpu.async_copy(a_hbm.at[isx[b]], ba[b], sa[b])
            pltpu.async_copy(
                ce_hbm.at[pl.ds((ec0 + c * k) * msg, k * msg)], bc[b], sc[b]
            )

        def gather_wait(c, b):
            pltpu.make_async_copy(a_hbm.at[isx[b]], ba[b], sa[b]).wait()
            pltpu.make_async_copy(
                ce_hbm.at[pl.ds((ec0 + c * k) * msg, k * msg)], bc[b], sc[b]
            ).wait()

        def _prime():
            idx_start(0, 0)
            idx_start(1, 1)

        pl.when(nch2 > 0)(_prime)

        def _bs_full():
            pltpu.async_copy(
                b_hbm.at[pl.ds(base_row * msg, wr * msg)],
                bstripe.at[pl.ds(0, wr * msg)],
                sa0,
            )

        def _bs_last():
            pltpu.async_copy(
                b_hbm.at[pl.ds(base_row * msg, last_rows * msg)],
                bstripe.at[pl.ds(0, last_rows * msg)],
                sa0,
            )

        pl.when(wid < NW - 1)(_bs_full)
        pl.when(wid == NW - 1)(_bs_last)

        def zvec(i, carry):
            outbuf[pl.ds(i * L, L)] = jnp.zeros((L,), jnp.float32)
            return carry

        lax.fori_loop(0, (wr + 1) * nvec, zvec, 0)

        def _bsw_full():
            pltpu.make_async_copy(
                b_hbm.at[pl.ds(base_row * msg, wr * msg)],
                bstripe.at[pl.ds(0, wr * msg)],
                sa0,
            ).wait()

        def _bsw_last():
            pltpu.make_async_copy(
                b_hbm.at[pl.ds(base_row * msg, last_rows * msg)],
                bstripe.at[pl.ds(0, last_rows * msg)],
                sa0,
            ).wait()

        pl.when(wid < NW - 1)(_bsw_full)
        pl.when(wid == NW - 1)(_bsw_last)

        def _first_gathers():
            idx_wait(0, 0)
            gather_start(0, 0)

        pl.when(nch2 > 0)(_first_gathers)

        zero16 = jnp.zeros((L,), jnp.float32)
        init = (jnp.array(-1, jnp.int32), (zero16,) * nvec)

        def process(c, b, carry, more):
            gather_wait(c, b)
            for j in range((k + L) // L):
                irc[b][pl.ds(j * L, L)] = irx[b][pl.ds(j * L, L)]
            pl.when(more)(lambda: idx_start(c + 2, b))

            def _next_gathers():
                idx_wait(c + 1, 1 - b)
                gather_start(c + 1, 1 - b)

            if b == 0:
                _next_gathers()
            else:
                pl.when(more)(_next_gathers)

            def one_edge(i, ecar):
                r_cur, accs = ecar
                e_abs = ec0 + c * k + i
                in_r = jnp.logical_and(e_abs >= e0, e_abs < e1)
                r_i = irc[b][pl.ds(i, L)][0]
                same = r_i == r_cur

                def flush():
                    rl_old = jnp.clip(
                        jnp.where(r_cur >= 0, r_cur - base_row, wr), 0, wr
                    )
                    for j in range(nvec):
                        outbuf[pl.ds(rl_old * msg + j * L, L)] = accs[j]

                pl.when(jnp.logical_and(in_r, jnp.logical_not(same)))(flush)
                r_new = jnp.where(in_r, r_i, r_cur)
                rl = jnp.clip(r_new - base_row, 0, wr - 1)
                new_accs = []
                for j in range(nvec):
                    m = jnp.maximum(
                        bc[b][pl.ds(i * msg + j * L, L)]
                        + ba[b][i, pl.ds(j * L, L)]
                        + bstripe[pl.ds(rl * msg + j * L, L)],
                        0.0,
                    )
                    a_prev = jnp.where(same, accs[j], 0.0)
                    new_accs.append(jnp.where(in_r, a_prev + m, accs[j]))
                return (r_new, tuple(new_accs))

            return lax.fori_loop(0, k, one_edge, carry)

        def pair(g, carry):
            more = g < nch2 - 1
            carry = process(2 * g, 0, carry, more)
            carry = process(2 * g + 1, 1, carry, more)
            return carry

        r_fin, accs_fin = lax.fori_loop(0, nch2, pair, init)
        rl_fin = jnp.clip(jnp.where(r_fin >= 0, r_fin - base_row, wr), 0, wr)
        for j in range(nvec):
            outbuf[pl.ds(rl_fin * msg + j * L, L)] = accs_fin[j]

        def _store_full():
            pltpu.sync_copy(
                outbuf.at[pl.ds(0, wr * msg)],
                out_hbm.at[pl.ds(base_row * msg, wr * msg)],
            )

        def _store_last():
            pltpu.sync_copy(
                outbuf.at[pl.ds(0, last_rows * msg)],
                out_hbm.at[pl.ds(base_row * msg, last_rows * msg)],
            )

        pl.when(wid < NW - 1)(_store_full)
        pl.when(wid == NW - 1)(_store_last)

    return sc_kernel(a_tab, b_flat, ce_flat, snd_p, rcv_p, bnd)


def _update(node_emb, msgs, w_upd, b_upd, *, bn):
    n, d = node_emb.shape
    msg = msgs.shape[-1]

    def body(x_ref, p_ref, w_ref, b_ref, o_ref):
        x = x_ref[...]
        p = p_ref[...]
        w1 = w_ref[:d]
        w2 = w_ref[d:]
        o_ref[...] = jnp.tanh(
            jnp.dot(x, w1, preferred_element_type=jnp.float32)
            + jnp.dot(p, w2, preferred_element_type=jnp.float32)
            + b_ref[...]
        )

    return pl.pallas_call(
        body,
        grid=(n // bn,),
        in_specs=[
            pl.BlockSpec((bn, d), lambda i: (i, 0)),
            pl.BlockSpec((bn, msg), lambda i: (i, 0)),
            pl.BlockSpec((d + msg, d), lambda i: (0, 0)),
            pl.BlockSpec((1, d), lambda i: (0, 0)),
        ],
        out_specs=pl.BlockSpec((bn, d), lambda i: (i, 0)),
        out_shape=jax.ShapeDtypeStruct((n, d), jnp.float32),
    )(node_emb, msgs, w_upd, b_upd)


def kernel(node_emb, edge_emb, senders, receivers, W_msg, b_msg, W_upd, b_upd):
    n, d = node_emb.shape
    e, de = edge_emb.shape
    msg = W_msg.shape[1]
    w_e = W_msg[:de]
    w_s = W_msg[de : de + d]
    w_r = W_msg[de + d :]

    k = 64
    wr = -(-n // NW)
    pad = 2000

    snd_p = jnp.concatenate([senders, jnp.zeros((pad,), jnp.int32)])
    rcv_p = jnp.concatenate([receivers, jnp.zeros((pad,), jnp.int32)])

    ce, a_tab, b_tab = _tables_and_edge_proj(
        node_emb, edge_emb, w_s, w_r, b_msg.reshape(1, msg), w_e,
        bn=2000, be=2000, e_out=e + pad,
    )

    bnd = jnp.searchsorted(
        receivers, (wr * jnp.arange(NW + 1)).astype(jnp.int32)
    ).astype(jnp.int32)
    bnd = jnp.concatenate([bnd, jnp.full((48 - (NW + 1),), e, jnp.int32)])

    msgs_flat = _sc_messages(
        a_tab, b_tab.reshape(-1), ce.reshape(-1), snd_p, rcv_p, bnd,
        n=n, msg=msg, k=k, wr=wr,
    )
    return _update(node_emb, msgs_flat.reshape(n, msg), W_upd, b_upd.reshape(1, d), bn=1000)


# speedup vs baseline: 1.0141x; 1.0011x over previous
import functools

import jax
import jax.numpy as jnp
from jax import lax
from jax.experimental import pallas as pl
from jax.experimental.pallas import tpu as pltpu
from jax.experimental.pallas import tpu_sc as plsc

NC, NS, L = 2, 16, 16
NW = NC * NS


def _tables_and_edge_proj(node_emb, edge_emb, w_s, w_r, b_msg, w_e, *, bn, be, e_out):
    n, d = node_emb.shape
    e, de = edge_emb.shape
    msg = w_e.shape[1]
    nreal = e // be
    nblk = n // bn

    def body(x_ref, we_ref, nd_ref, ws_ref, wr_ref, b_ref, o_ref, a_ref, bb_ref):
        i = pl.program_id(0)
        o_ref[...] = jnp.dot(x_ref[...], we_ref[...], preferred_element_type=jnp.float32)

        @pl.when(i < nblk)
        def _():
            x = nd_ref[...]
            a_ref[...] = jnp.dot(x, ws_ref[...], preferred_element_type=jnp.float32)
            bb_ref[...] = (
                jnp.dot(x, wr_ref[...], preferred_element_type=jnp.float32)
                + b_ref[...]
            )

    clamp_n = lambda i: (jnp.minimum(i, nblk - 1), 0)
    return pl.pallas_call(
        body,
        grid=(e_out // be,),
        in_specs=[
            pl.BlockSpec((be, de), lambda i: (jnp.minimum(i, nreal - 1), 0)),
            pl.BlockSpec((de, msg), lambda i: (0, 0)),
            pl.BlockSpec((bn, d), clamp_n),
            pl.BlockSpec((d, msg), lambda i: (0, 0)),
            pl.BlockSpec((d, msg), lambda i: (0, 0)),
            pl.BlockSpec((1, msg), lambda i: (0, 0)),
        ],
        out_specs=[
            pl.BlockSpec((be, msg), lambda i: (i, 0)),
            pl.BlockSpec((bn, msg), clamp_n),
            pl.BlockSpec((bn, msg), clamp_n),
        ],
        out_shape=[
            jax.ShapeDtypeStruct((e_out, msg), jnp.float32),
            jax.ShapeDtypeStruct((n, msg), jnp.float32),
            jax.ShapeDtypeStruct((n, msg), jnp.float32),
        ],
    )(edge_emb, w_e, node_emb, w_s, w_r, b_msg)


def _sc_messages(a_tab, b_flat, ce_flat, snd_p, rcv_p, bnd, *, n, msg, k, wr):
    nvec = msg // L
    last_rows = n - wr * (NW - 1)
    assert 0 < last_rows <= wr
    mesh = plsc.VectorSubcoreMesh(core_axis_name="c", subcore_axis_name="s")

    @functools.partial(
        pl.kernel,
        out_type=jax.ShapeDtypeStruct((n * msg,), jnp.float32),
        mesh=mesh,
        scratch_types=[
            pltpu.VMEM(((wr + 1) * msg,), jnp.float32),
            pltpu.VMEM((wr * msg,), jnp.float32),
            pltpu.VMEM((k, msg), jnp.float32),
            pltpu.VMEM((k * msg,), jnp.float32),
            pltpu.VMEM((k,), jnp.int32),
            pltpu.VMEM((k + L,), jnp.int32),
            pltpu.VMEM((k + L,), jnp.int32),
            pltpu.VMEM((k, msg), jnp.float32),
            pltpu.VMEM((k * msg,), jnp.float32),
            pltpu.VMEM((k,), jnp.int32),
            pltpu.VMEM((k + L,), jnp.int32),
            pltpu.VMEM((k + L,), jnp.int32),
            pltpu.VMEM((48,), jnp.int32),
            pltpu.SemaphoreType.DMA,
            pltpu.SemaphoreType.DMA,
            pltpu.SemaphoreType.DMA,
            pltpu.SemaphoreType.DMA,
            pltpu.SemaphoreType.DMA,
            pltpu.SemaphoreType.DMA,
        ],
    )
    def sc_kernel(
        a_hbm, b_hbm, ce_hbm, snd_hbm, rcv_hbm, bnd_hbm, out_hbm,
        outbuf, bstripe,
        ba0, bc0, is0, ir0, irc0, ba1, bc1, is1, ir1, irc1, bndv,
        sa0, sc0, si0, sa1, sc1, si1,
    ):
        cid = lax.axis_index("c")
        sid = lax.axis_index("s")
        wid = sid * NC + cid
        ba, bc = (ba0, ba1), (bc0, bc1)
        isx, irx, irc = (is0, is1), (ir0, ir1), (irc0, irc1)
        sa, sc, si = (sa0, sa1), (sc0, sc1), (si0, si1)

        pltpu.sync_copy(bnd_hbm, bndv)
        e0 = bndv[pl.ds(wid, L)][0]
        e1 = bndv[pl.ds(wid + 1, L)][0]
        base_row = wid * wr
        ec0 = (e0 // 8) * 8
        nch2 = (e1 - ec0 + 2 * k - 1) // (2 * k)

        def idx_start(c, b):
            off = ec0 + c * k
            pltpu.async_copy(snd_hbm.at[pl.ds(off, k)], isx[b], si[b])
            pltpu.async_copy(rcv_hbm.at[pl.ds(off, k)], irx[b].at[pl.ds(0, k)], si[b])

        def idx_wait(c, b):
            off = ec0 + c * k
            pltpu.make_async_copy(snd_hbm.at[pl.ds(off, k)], isx[b], si[b]).wait()
            pltpu.make_async_copy(
                rcv_hbm.at[pl.ds(off, k)], irx[b].at[pl.ds(0, k)], si[b]
            ).wait()

        def gather_start(c, b):
            pltpu.async_copy(a_hbm.at[isx[b]], ba[b], sa[b])
            pltpu.async_copy(
                ce_hbm.at[pl.ds((ec0 + c * k) * msg, k * msg)], bc[b], sc[b]
            )

        def gather_wait(c, b):
            pltpu.make_async_copy(a_hbm.at[isx[b]], ba[b], sa[b]).wait()
            pltpu.make_async_copy(
                ce_hbm.at[pl.ds((ec0 + c * k) * msg, k * msg)], bc[b], sc[b]
            ).wait()

        def _prime():
            idx_start(0, 0)
            idx_start(1, 1)

        pl.when(nch2 > 0)(_prime)

        def _bs_full():
            pltpu.async_copy(
                b_hbm.at[pl.ds(base_row * msg, wr * msg)],
                bstripe.at[pl.ds(0, wr * msg)],
                sa0,
            )

        def _bs_last():
            pltpu.async_copy(
                b_hbm.at[pl.ds(base_row * msg, last_rows * msg)],
                bstripe.at[pl.ds(0, last_rows * msg)],
                sa0,
            )

        pl.when(wid < NW - 1)(_bs_full)
        pl.when(wid == NW - 1)(_bs_last)

        def zvec(i, carry):
            outbuf[pl.ds(i * L, L)] = jnp.zeros((L,), jnp.float32)
            return carry

        lax.fori_loop(0, (wr + 1) * nvec, zvec, 0)

        def _bsw_full():
            pltpu.make_async_copy(
                b_hbm.at[pl.ds(base_row * msg, wr * msg)],
                bstripe.at[pl.ds(0, wr * msg)],
                sa0,
            ).wait()

        def _bsw_last():
            pltpu.make_async_copy(
                b_hbm.at[pl.ds(base_row * msg, last_rows * msg)],
                bstripe.at[pl.ds(0, last_rows * msg)],
                sa0,
            ).wait()

        pl.when(wid < NW - 1)(_bsw_full)
        pl.when(wid == NW - 1)(_bsw_last)

        def _first_gathers():
            idx_wait(0, 0)
            gather_start(0, 0)

        pl.when(nch2 > 0)(_first_gathers)

        zero16 = jnp.zeros((L,), jnp.float32)
        init = (jnp.array(-1, jnp.int32), (zero16,) * nvec)

        def process(c, b, carry, more):
            gather_wait(c, b)
            for j in range((k + L) // L):
                irc[b][pl.ds(j * L, L)] = irx[b][pl.ds(j * L, L)]
            pl.when(more)(lambda: idx_start(c + 2, b))

            def _next_gathers():
                idx_wait(c + 1, 1 - b)
                gather_start(c + 1, 1 - b)

            if b == 0:
                _next_gathers()
            else:
                pl.when(more)(_next_gathers)

            def one_edge(i, ecar):
                r_cur, accs = ecar
                e_abs = ec0 + c * k + i
                in_r = jnp.logical_and(e_abs >= e0, e_abs < e1)
                r_i = irc[b][pl.ds(i, L)][0]
                same = r_i == r_cur

                def flush():
                    rl_old = jnp.clip(
                        jnp.where(r_cur >= 0, r_cur - base_row, wr), 0, wr
                    )
                    for j in range(nvec):
                        outbuf[pl.ds(rl_old * msg + j * L, L)] = accs[j]

                pl.when(jnp.logical_and(in_r, jnp.logical_not(same)))(flush)
                r_new = jnp.where(in_r, r_i, r_cur)
                rl = jnp.clip(r_new - base_row, 0, wr - 1)
                new_accs = []
                for j in range(nvec):
                    m = jnp.maximum(
                        bc[b][pl.ds(i * msg + j * L, L)]
                        + ba[b][i, pl.ds(j * L, L)]
                        + bstripe[pl.ds(rl * msg + j * L, L)],
                        0.0,
                    )
                    a_prev = jnp.where(same, accs[j], 0.0)
                    new_accs.append(jnp.where(in_r, a_prev + m, accs[j]))
                return (r_new, tuple(new_accs))

            return lax.fori_loop(0, k, one_edge, carry)

        def pair(g, carry):
            more = g < nch2 - 1
            carry = process(2 * g, 0, carry, more)
            carry = process(2 * g + 1, 1, carry, more)
            return carry

        r_fin, accs_fin = lax.fori_loop(0, nch2, pair, init)
        rl_fin = jnp.clip(jnp.where(r_fin >= 0, r_fin - base_row, wr), 0, wr)
        for j in range(nvec):
            outbuf[pl.ds(rl_fin * msg + j * L, L)] = accs_fin[j]

        def _store_full():
            pltpu.sync_copy(
                outbuf.at[pl.ds(0, wr * msg)],
                out_hbm.at[pl.ds(base_row * msg, wr * msg)],
            )

        def _store_last():
            pltpu.sync_copy(
                outbuf.at[pl.ds(0, last_rows * msg)],
                out_hbm.at[pl.ds(base_row * msg, last_rows * msg)],
            )

        pl.when(wid < NW - 1)(_store_full)
        pl.when(wid == NW - 1)(_store_last)

    return sc_kernel(a_tab, b_flat, ce_flat, snd_p, rcv_p, bnd)


def _update(node_emb, msgs, w_upd, b_upd, *, bn):
    n, d = node_emb.shape
    msg = msgs.shape[-1]

    def body(x_ref, p_ref, w_ref, b_ref, o_ref):
        x = x_ref[...]
        p = p_ref[...]
        w1 = w_ref[:d]
        w2 = w_ref[d:]
        o_ref[...] = jnp.tanh(
            jnp.dot(x, w1, preferred_element_type=jnp.float32)
            + jnp.dot(p, w2, preferred_element_type=jnp.float32)
            + b_ref[...]
        )

    return pl.pallas_call(
        body,
        grid=(n // bn,),
        in_specs=[
            pl.BlockSpec((bn, d), lambda i: (i, 0)),
            pl.BlockSpec((bn, msg), lambda i: (i, 0)),
            pl.BlockSpec((d + msg, d), lambda i: (0, 0)),
            pl.BlockSpec((1, d), lambda i: (0, 0)),
        ],
        out_specs=pl.BlockSpec((bn, d), lambda i: (i, 0)),
        out_shape=jax.ShapeDtypeStruct((n, d), jnp.float32),
    )(node_emb, msgs, w_upd, b_upd)


def kernel(node_emb, edge_emb, senders, receivers, W_msg, b_msg, W_upd, b_upd):
    n, d = node_emb.shape
    e, de = edge_emb.shape
    msg = W_msg.shape[1]
    w_e = W_msg[:de]
    w_s = W_msg[de : de + d]
    w_r = W_msg[de + d :]

    k = 64
    wr = -(-n // NW)
    pad = 2000

    snd_p = jnp.concatenate([senders, jnp.zeros((pad,), jnp.int32)])
    rcv_p = jnp.concatenate([receivers, jnp.zeros((pad,), jnp.int32)])

    ce, a_tab, b_tab = _tables_and_edge_proj(
        node_emb, edge_emb, w_s, w_r, b_msg.reshape(1, msg), w_e,
        bn=2000, be=2000, e_out=e + pad,
    )

    targets = (wr * jnp.arange(NW + 1)).astype(jnp.int32)
    bnd = jnp.sum(
        receivers[None, :] < targets[:, None], axis=1, dtype=jnp.int32
    )
    bnd = jnp.concatenate([bnd, jnp.full((48 - (NW + 1),), e, jnp.int32)])

    msgs_flat = _sc_messages(
        a_tab, b_tab.reshape(-1), ce.reshape(-1), snd_p, rcv_p, bnd,
        n=n, msg=msg, k=k, wr=wr,
    )
    return _update(node_emb, msgs_flat.reshape(n, msg), W_upd, b_upd.reshape(1, d), bn=1000)
